# baseline (speedup 1.0000x reference)
import jax
import jax.numpy as jnp
from jax.experimental import pallas as pl
from jax.experimental.pallas import tpu as pltpu

_VMEM_LIMIT = 48 * 1024 * 1024
_BT = 1024
_SB1 = 64
_SB2 = 128


def _round_up(n, m):
    return ((n + m - 1) // m) * m


def _fused_kernel(x_ref, t1_ref, cb1_ref, t2_ref, cb2_ref,
                  w1_ref, fb1_ref, w2_ref, fb2_ref, w3_ref, fb3_ref,
                  o_ref, p1_ref, s2_ref):
    bt = x_ref.shape[0]

    for c in range(bt // _SB1):
        lhs = jnp.concatenate(
            [jnp.pad(x_ref[c * _SB1:(c + 1) * _SB1, pl.ds(i, 12, stride=2), :]
                     .astype(jnp.bfloat16), ((0, 0), (0, 0), (0, 4)))
             for i in range(6)], axis=2)
        lhs = lhs.reshape(_SB1 * 12, 192)
        acc = jnp.dot(lhs, t1_ref[...], preferred_element_type=jnp.float32)
        y = acc + cb1_ref[...]
        y = jnp.maximum(y[:, :256], y[:, 256:])
        y = jnp.maximum(y[:, :128], y[:, 128:])
        y = jnp.maximum(y, 0.0)
        p1_ref[c * _SB1:(c + 1) * _SB1] = y.reshape(_SB1, 12, 128)

    for c in range(bt // _SB2):
        lhs = jnp.concatenate(
            [p1_ref[c * _SB2:(c + 1) * _SB2, pl.ds(i, 4, stride=2), :]
             for i in range(6)], axis=2)
        lhs = lhs.reshape(_SB2 * 4, 768).astype(jnp.bfloat16)
        acc = jnp.dot(lhs, t2_ref[...], preferred_element_type=jnp.float32)
        y = acc + cb2_ref[...]
        y = jnp.maximum(y[:, :256], y[:, 256:])
        y = jnp.maximum(y[:, :128], y[:, 128:])
        y = jnp.maximum(y, 0.0)
        s2_ref[c * _SB2 * 4:(c + 1) * _SB2 * 4, :] = y

    feat = jnp.concatenate(
        [s2_ref[pl.ds(h, bt, stride=4), :] for h in range(4)],
        axis=1).astype(jnp.bfloat16)
    h1 = jnp.dot(feat, w1_ref[...], preferred_element_type=jnp.float32)
    h1 = jnp.maximum(h1 + fb1_ref[...], 0.0)

    g = jnp.dot(h1.astype(jnp.bfloat16), w2_ref[...],
                preferred_element_type=jnp.float32)
    g = jnp.maximum(g + fb2_ref[...], 0.0)
    out = jnp.dot(g.astype(jnp.bfloat16), w3_ref[...],
                  preferred_element_type=jnp.float32) + fb3_ref[...]
    o_ref[...] = out.reshape(1, bt, 128)


def _shift_pair(t):
    pad = jnp.zeros_like(t[:1])
    z0 = jnp.concatenate([t, pad], axis=0)
    z1 = jnp.concatenate([pad, t], axis=0)
    return jnp.concatenate([z0, z1], axis=2)


def kernel(t1, cb1, t2, cb2, w1, fb1, w2, fb2, w3, fb3, x):
    B = x.shape[0]
    xs = x.reshape(B, 28, 28).astype(jnp.float32)
    bt = _BT
    Bp = _round_up(B, bt)
    if Bp != B:
        xs = jnp.pad(xs, ((0, Bp - B), (0, 0), (0, 0)))
    grid = Bp // bt

    t1r = jnp.pad(_shift_pair(t1), ((0, 0), (0, 4), (0, 0))).reshape(192, 512)
    cb1r = jnp.concatenate([cb1, cb1], axis=1)
    t2r = _shift_pair(t2).reshape(768, 512)
    cb2r = jnp.concatenate([cb2, cb2], axis=1)
    w1r = w1.reshape(512, 128)

    def whole(a):
        nd = a.ndim
        return pl.BlockSpec(a.shape, lambda i, _nd=nd: (0,) * _nd)

    out = pl.pallas_call(
        _fused_kernel,
        out_shape=jax.ShapeDtypeStruct((grid, bt, 128), jnp.float32),
        grid=(grid,),
        in_specs=[
            pl.BlockSpec((bt, 28, 28), lambda i: (i, 0, 0)),
            whole(t1r), whole(cb1r),
            whole(t2r), whole(cb2r),
            whole(w1r), whole(fb1),
            whole(w2), whole(fb2),
            whole(w3), whole(fb3),
        ],
        out_specs=pl.BlockSpec((1, bt, 128), lambda i: (i, 0, 0)),
        scratch_shapes=[
            pltpu.VMEM((bt, 12, 128), jnp.float32),
            pltpu.VMEM((bt * 4, 128), jnp.float32),
        ],
        compiler_params=pltpu.CompilerParams(
            dimension_semantics=("parallel",),
            vmem_limit_bytes=_VMEM_LIMIT,
        ),
    )(xs, t1r, cb1r, t2r, cb2r, w1r, fb1, w2, fb2, w3, fb3)

    return out.reshape(Bp, 128)[:B, :10]


# speedup vs baseline: 1.0409x; 1.0409x over previous
import jax
import jax.numpy as jnp
from jax.experimental import pallas as pl
from jax.experimental.pallas import tpu as pltpu

_VMEM_LIMIT = 48 * 1024 * 1024
_BT = 1024
_SB1 = 64
_SB2 = 128


def _round_up(n, m):
    return ((n + m - 1) // m) * m


def _fused_kernel(x_ref, t1_ref, cb1_ref, t2_ref, cb2_ref,
                  w1_ref, fb1_ref, w2_ref, fb2_ref, w3_ref, fb3_ref,
                  o_ref, p1_ref, s2_ref):
    bt = x_ref.shape[0]

    for c in range(bt // _SB1):
        lhs = jnp.concatenate(
            [x_ref[c * _SB1:(c + 1) * _SB1, pl.ds(i, 12, stride=2), :]
             .astype(jnp.bfloat16) for i in range(6)], axis=2)
        lhs = lhs.reshape(_SB1 * 12, 168)
        acc = jnp.dot(lhs, t1_ref[...], preferred_element_type=jnp.float32)
        y = jnp.maximum(acc[:, :256], acc[:, 256:]) + cb1_ref[...]
        y = jnp.maximum(y[:, :128], y[:, 128:])
        y = jnp.maximum(y, 0.0)
        p1_ref[c * _SB1:(c + 1) * _SB1] = y.reshape(_SB1, 12, 128)

    for c in range(bt // _SB2):
        lhs = jnp.concatenate(
            [p1_ref[c * _SB2:(c + 1) * _SB2, pl.ds(i, 4, stride=2), :]
             for i in range(6)], axis=2)
        lhs = lhs.reshape(_SB2 * 4, 768).astype(jnp.bfloat16)
        acc = jnp.dot(lhs, t2_ref[...], preferred_element_type=jnp.float32)
        y = jnp.maximum(acc[:, :256], acc[:, 256:]) + cb2_ref[...]
        y = jnp.maximum(y[:, :128], y[:, 128:])
        y = jnp.maximum(y, 0.0)
        s2_ref[c * _SB2 * 4:(c + 1) * _SB2 * 4, :] = y

    feat = jnp.concatenate(
        [s2_ref[pl.ds(h, bt, stride=4), :] for h in range(4)],
        axis=1).astype(jnp.bfloat16)
    h1 = jnp.dot(feat, w1_ref[...], preferred_element_type=jnp.float32)
    h1 = jnp.maximum(h1 + fb1_ref[...], 0.0)

    g = jnp.dot(h1.astype(jnp.bfloat16), w2_ref[...],
                preferred_element_type=jnp.float32)
    g = jnp.maximum(g + fb2_ref[...], 0.0)
    out = jnp.dot(g.astype(jnp.bfloat16), w3_ref[...],
                  preferred_element_type=jnp.float32) + fb3_ref[...]
    o_ref[...] = out.reshape(1, bt, 128)


def _shift_pair(t):
    pad = jnp.zeros_like(t[:1])
    z0 = jnp.concatenate([t, pad], axis=0)
    z1 = jnp.concatenate([pad, t], axis=0)
    return jnp.concatenate([z0, z1], axis=2)


def kernel(t1, cb1, t2, cb2, w1, fb1, w2, fb2, w3, fb3, x):
    B = x.shape[0]
    xs = x.reshape(B, 28, 28).astype(jnp.float32)
    bt = _BT
    Bp = _round_up(B, bt)
    if Bp != B:
        xs = jnp.pad(xs, ((0, Bp - B), (0, 0), (0, 0)))
    grid = Bp // bt

    t1r = _shift_pair(t1).reshape(168, 512)
    cb1r = cb1
    t2r = _shift_pair(t2).reshape(768, 512)
    cb2r = cb2
    w1r = w1.reshape(512, 128)

    def whole(a):
        nd = a.ndim
        return pl.BlockSpec(a.shape, lambda i, _nd=nd: (0,) * _nd)

    out = pl.pallas_call(
        _fused_kernel,
        out_shape=jax.ShapeDtypeStruct((grid, bt, 128), jnp.float32),
        grid=(grid,),
        in_specs=[
            pl.BlockSpec((bt, 28, 28), lambda i: (i, 0, 0)),
            whole(t1r), whole(cb1r),
            whole(t2r), whole(cb2r),
            whole(w1r), whole(fb1),
            whole(w2), whole(fb2),
            whole(w3), whole(fb3),
        ],
        out_specs=pl.BlockSpec((1, bt, 128), lambda i: (i, 0, 0)),
        scratch_shapes=[
            pltpu.VMEM((bt, 12, 128), jnp.float32),
            pltpu.VMEM((bt * 4, 128), jnp.float32),
        ],
        compiler_params=pltpu.CompilerParams(
            dimension_semantics=("parallel",),
            vmem_limit_bytes=_VMEM_LIMIT,
        ),
    )(xs, t1r, cb1r, t2r, cb2r, w1r, fb1, w2, fb2, w3, fb3)

    return out.reshape(Bp, 128)[:B, :10]
